# initial kernel scaffold (unmeasured)
import functools

import jax
import jax.numpy as jnp
from jax import lax
from jax.experimental import pallas as pl
from jax.experimental.pallas import tpu as pltpu

N_DEV = 16


def kernel(x, w_mat):
    m_loc, k = x.shape
    _, n = w_mat.shape
    n_loc = n // N_DEV
    m = m_loc * N_DEV

    def body(x_ref, w_hbm, out_ref,
             w_vmem, send_buf, recv_buf, amax_buf,
             w_sems, send_sems, recv_sems, amax_send_sems, amax_recv_sems):
        my_pos = lax.axis_index("i")

        barrier = pltpu.get_barrier_semaphore()
        for p in range(N_DEV):
            pl.semaphore_signal(
                barrier, inc=1,
                device_id=(p,), device_id_type=pl.DeviceIdType.MESH,
            )
        pl.semaphore_wait(barrier, N_DEV)

        def w_copy(s):
            dest = (my_pos + s) % N_DEV
            return pltpu.make_async_copy(
                w_hbm.at[:, pl.ds(dest * n_loc, n_loc)],
                w_vmem.at[s % 2],
                w_sems.at[s % 2],
            )

        w_copy(0).start()

        data_rdmas = []
        amax_parts = []
        for s in range(N_DEV):
            w_copy(s).wait()
            if s + 1 < N_DEV:
                w_copy(s + 1).start()
            y = jnp.dot(x_ref[:, :], w_vmem[s % 2],
                        preferred_element_type=jnp.float32)
            amax_parts.append(jnp.max(jnp.abs(y)))
            if s == 0:
                recv_buf[0] = y
            else:
                send_buf[s] = y
                dest = (my_pos + s) % N_DEV
                rdma = pltpu.make_async_remote_copy(
                    src_ref=send_buf.at[s],
                    dst_ref=recv_buf.at[s],
                    send_sem=send_sems.at[s],
                    recv_sem=recv_sems.at[s],
                    device_id=(dest,),
                    device_id_type=pl.DeviceIdType.MESH,
                )
                rdma.start()
                data_rdmas.append(rdma)

        my_amax = functools.reduce(jnp.maximum, amax_parts)
        amax_buf[0] = jnp.full((8, 128), my_amax, jnp.float32)
        amax_rdmas = []
        for s in range(1, N_DEV):
            dest = (my_pos + s) % N_DEV
            r = pltpu.make_async_remote_copy(
                src_ref=amax_buf.at[0],
                dst_ref=amax_buf.at[s],
                send_sem=amax_send_sems.at[s],
                recv_sem=amax_recv_sems.at[s],
                device_id=(dest,),
                device_id_type=pl.DeviceIdType.MESH,
            )
            r.start()
            amax_rdmas.append(r)

        for r in amax_rdmas:
            r.wait_recv()
        global_amax = jnp.max(amax_buf[:, :, :])
        scale = global_amax / 127.0

        for s in range(N_DEV):
            if s > 0:
                data_rdmas[s - 1].wait_recv()
            y = recv_buf[s]
            q = jnp.clip(jnp.round(y / scale), -127.0, 127.0)
            src = (my_pos - s) % N_DEV
            out_ref[pl.ds(src * m_loc, m_loc), :] = q * scale

        for r in data_rdmas:
            r.wait_send()
        for r in amax_rdmas:
            r.wait_send()

    return pl.pallas_call(
        body,
        out_shape=jax.ShapeDtypeStruct((m, n_loc), jnp.float32),
        in_specs=[
            pl.BlockSpec(memory_space=pltpu.VMEM),
            pl.BlockSpec(memory_space=pltpu.ANY),
        ],
        out_specs=pl.BlockSpec(memory_space=pltpu.VMEM),
        scratch_shapes=[
            pltpu.VMEM((2, k, n_loc), jnp.float32),
            pltpu.VMEM((N_DEV, m_loc, n_loc), jnp.float32),
            pltpu.VMEM((N_DEV, m_loc, n_loc), jnp.float32),
            pltpu.VMEM((N_DEV, 8, 128), jnp.float32),
            pltpu.SemaphoreType.DMA((2,)),
            pltpu.SemaphoreType.DMA((N_DEV,)),
            pltpu.SemaphoreType.DMA((N_DEV,)),
            pltpu.SemaphoreType.DMA((N_DEV,)),
            pltpu.SemaphoreType.DMA((N_DEV,)),
        ],
        compiler_params=pltpu.CompilerParams(collective_id=0),
    )(x, w_mat)


# baseline (device time: 135366 ns/iter reference)
import functools

import jax
import jax.numpy as jnp
from jax import lax
from jax.experimental import pallas as pl
from jax.experimental.pallas import tpu as pltpu

N_DEV = 16


def kernel(x, w_mat):
    m_loc, k = x.shape
    _, n = w_mat.shape
    n_loc = n // N_DEV
    m = m_loc * N_DEV

    def body(x_ref, w_hbm, out_ref,
             w_vmem, send_buf, recv_buf, amax_buf,
             w_sems, send_sems, recv_sems, amax_send_sems, amax_recv_sems):
        my_pos = lax.axis_index("i")

        barrier = pltpu.get_barrier_semaphore()
        for p in range(N_DEV):
            pl.semaphore_signal(
                barrier, inc=1,
                device_id=(p,), device_id_type=pl.DeviceIdType.MESH,
            )
        pl.semaphore_wait(barrier, N_DEV)

        def w_copy(s):
            dest = (my_pos + s) % N_DEV
            return pltpu.make_async_copy(
                w_hbm.at[:, pl.ds(dest * n_loc, n_loc)],
                w_vmem.at[s % 2],
                w_sems.at[s % 2],
            )

        w_copy(0).start()

        data_rdmas = []
        amax_parts = []
        for s in range(N_DEV):
            w_copy(s).wait()
            if s + 1 < N_DEV:
                w_copy(s + 1).start()
            y = jnp.dot(x_ref[:, :], w_vmem[s % 2],
                        preferred_element_type=jnp.float32)
            amax_parts.append(jnp.max(jnp.abs(y)))
            if s == 0:
                recv_buf[0] = y
            else:
                send_buf[s] = y
                dest = (my_pos + s) % N_DEV
                rdma = pltpu.make_async_remote_copy(
                    src_ref=send_buf.at[s],
                    dst_ref=recv_buf.at[s],
                    send_sem=send_sems.at[s],
                    recv_sem=recv_sems.at[s],
                    device_id=(dest,),
                    device_id_type=pl.DeviceIdType.MESH,
                )
                rdma.start()
                data_rdmas.append(rdma)

        my_amax = functools.reduce(jnp.maximum, amax_parts)
        amax_buf[0] = jnp.full((8, 128), my_amax, jnp.float32)
        amax_rdmas = []
        for s in range(1, N_DEV):
            dest = (my_pos + s) % N_DEV
            r = pltpu.make_async_remote_copy(
                src_ref=amax_buf.at[0],
                dst_ref=amax_buf.at[s],
                send_sem=amax_send_sems.at[s],
                recv_sem=amax_recv_sems.at[s],
                device_id=(dest,),
                device_id_type=pl.DeviceIdType.MESH,
            )
            r.start()
            amax_rdmas.append(r)

        for r in amax_rdmas:
            r.wait_recv()
        global_amax = jnp.max(amax_buf[:, :, :])
        scale = global_amax / 127.0

        for s in range(N_DEV):
            if s > 0:
                data_rdmas[s - 1].wait_recv()
            y = recv_buf[s]
            q = jnp.clip(jnp.round(y / scale), -127.0, 127.0)
            src = (my_pos - s) % N_DEV
            out_ref[pl.ds(src * m_loc, m_loc), :] = q * scale

        for r in data_rdmas:
            r.wait_send()
        for r in amax_rdmas:
            r.wait_send()

    return pl.pallas_call(
        body,
        out_shape=jax.ShapeDtypeStruct((m, n_loc), jnp.float32),
        in_specs=[
            pl.BlockSpec(memory_space=pltpu.VMEM),
            pl.BlockSpec(memory_space=pl.ANY),
        ],
        out_specs=pl.BlockSpec(memory_space=pltpu.VMEM),
        scratch_shapes=[
            pltpu.VMEM((2, k, n_loc), jnp.float32),
            pltpu.VMEM((N_DEV, m_loc, n_loc), jnp.float32),
            pltpu.VMEM((N_DEV, m_loc, n_loc), jnp.float32),
            pltpu.VMEM((N_DEV, 8, 128), jnp.float32),
            pltpu.SemaphoreType.DMA((2,)),
            pltpu.SemaphoreType.DMA((N_DEV,)),
            pltpu.SemaphoreType.DMA((N_DEV,)),
            pltpu.SemaphoreType.DMA((N_DEV,)),
            pltpu.SemaphoreType.DMA((N_DEV,)),
        ],
        compiler_params=pltpu.CompilerParams(
            collective_id=0,
            vmem_limit_bytes=100 * 1024 * 1024,
        ),
    )(x, w_mat)


# device time: 89346 ns/iter; 1.5151x vs baseline; 1.5151x over previous
import functools

import jax
import jax.numpy as jnp
from jax import lax
from jax.experimental import pallas as pl
from jax.experimental.pallas import tpu as pltpu

N_DEV = 16
N_WSLOT = 2
N_WCHUNK = 4


def kernel(x, w_mat):
    m_loc, k = x.shape
    _, n = w_mat.shape
    n_loc = n // N_DEV
    m = m_loc * N_DEV
    rows = k // N_WCHUNK

    def body(x_ref, w_hbm, out_ref,
             w_vmem, blocks, send_i8, recv_i8, amax_buf,
             w_sems, send_sems, recv_sems, amax_send_sems, amax_recv_sems):
        my_pos = lax.axis_index("i")

        barrier = pltpu.get_barrier_semaphore()
        for p in range(N_DEV):
            pl.semaphore_signal(
                barrier, inc=1,
                device_id=(p,), device_id_type=pl.DeviceIdType.MESH,
            )
        pl.semaphore_wait(barrier, N_DEV)

        def w_copies(s):
            dest = (my_pos + s) % N_DEV
            return [
                pltpu.make_async_copy(
                    w_hbm.at[pl.ds(c * rows, rows), pl.ds(dest * n_loc, n_loc)],
                    w_vmem.at[s % N_WSLOT, pl.ds(c * rows, rows)],
                    w_sems.at[s % N_WSLOT, c],
                )
                for c in range(N_WCHUNK)
            ]

        for s in range(N_WSLOT):
            for cp in w_copies(s):
                cp.start()
        amax_parts = []
        for s in range(N_DEV):
            for cp in w_copies(s):
                cp.wait()
            if s + N_WSLOT < N_DEV:
                for cp in w_copies(s + N_WSLOT):
                    cp.start()
            y = jnp.dot(x_ref[:, :], w_vmem[s % N_WSLOT],
                        preferred_element_type=jnp.float32)
            amax_parts.append(jnp.max(jnp.abs(y)))
            blocks[s] = y

        my_amax = functools.reduce(jnp.maximum, amax_parts)
        amax_buf[0] = jnp.full((8, 128), my_amax, jnp.float32)
        amax_rdmas = []
        for s in range(1, N_DEV):
            dest = (my_pos + s) % N_DEV
            r = pltpu.make_async_remote_copy(
                src_ref=amax_buf.at[0],
                dst_ref=amax_buf.at[s],
                send_sem=amax_send_sems.at[s],
                recv_sem=amax_recv_sems.at[s],
                device_id=(dest,),
                device_id_type=pl.DeviceIdType.MESH,
            )
            r.start()
            amax_rdmas.append(r)
        for r in amax_rdmas:
            r.wait_recv()
        global_amax = jnp.max(amax_buf[:, :, :])
        scale = global_amax / 127.0

        def quant(y):
            return jnp.clip(jnp.round(y / scale), -127.0, 127.0).astype(jnp.int8)

        data_rdmas = []
        for s in range(N_DEV):
            if s == 0:
                recv_i8[0] = quant(blocks[0])
            else:
                send_i8[s] = quant(blocks[s])
                dest = (my_pos + s) % N_DEV
                rdma = pltpu.make_async_remote_copy(
                    src_ref=send_i8.at[s],
                    dst_ref=recv_i8.at[s],
                    send_sem=send_sems.at[s],
                    recv_sem=recv_sems.at[s],
                    device_id=(dest,),
                    device_id_type=pl.DeviceIdType.MESH,
                )
                rdma.start()
                data_rdmas.append(rdma)

        for s in range(N_DEV):
            if s > 0:
                data_rdmas[s - 1].wait_recv()
            src = (my_pos - s) % N_DEV
            out_ref[pl.ds(src * m_loc, m_loc), :] = (
                recv_i8[s].astype(jnp.float32) * scale)

        for r in data_rdmas:
            r.wait_send()
        for r in amax_rdmas:
            r.wait_send()

    return pl.pallas_call(
        body,
        out_shape=jax.ShapeDtypeStruct((m, n_loc), jnp.float32),
        in_specs=[
            pl.BlockSpec(memory_space=pltpu.VMEM),
            pl.BlockSpec(memory_space=pl.ANY),
        ],
        out_specs=pl.BlockSpec(memory_space=pltpu.VMEM),
        scratch_shapes=[
            pltpu.VMEM((N_WSLOT, k, n_loc), jnp.float32),
            pltpu.VMEM((N_DEV, m_loc, n_loc), jnp.float32),
            pltpu.VMEM((N_DEV, m_loc, n_loc), jnp.int8),
            pltpu.VMEM((N_DEV, m_loc, n_loc), jnp.int8),
            pltpu.VMEM((N_DEV, 8, 128), jnp.float32),
            pltpu.SemaphoreType.DMA((N_WSLOT, N_WCHUNK)),
            pltpu.SemaphoreType.DMA((N_DEV,)),
            pltpu.SemaphoreType.DMA((N_DEV,)),
            pltpu.SemaphoreType.DMA((N_DEV,)),
            pltpu.SemaphoreType.DMA((N_DEV,)),
        ],
        compiler_params=pltpu.CompilerParams(
            collective_id=0,
            vmem_limit_bytes=100 * 1024 * 1024,
        ),
    )(x, w_mat)


# device time: 79510 ns/iter; 1.7025x vs baseline; 1.1237x over previous
import functools

import jax
import jax.numpy as jnp
from jax import lax
from jax.experimental import pallas as pl
from jax.experimental.pallas import tpu as pltpu

N_DEV = 16
N_WSLOT = 2
N_WCHUNK = 4
SPLIT = 6


def kernel(x, w_mat):
    m_loc, k = x.shape
    _, n = w_mat.shape
    n_loc = n // N_DEV
    m = m_loc * N_DEV
    rows = k // N_WCHUNK

    def body(x_ref, w_hbm, out_ref,
             w_vmem, blocks, recv_f32, send_i8, recv_i8, amax_buf,
             w_sems, send_sems, recv_sems, amax_send_sems, amax_recv_sems):
        my_pos = lax.axis_index("i")

        barrier = pltpu.get_barrier_semaphore()
        for p in range(N_DEV):
            pl.semaphore_signal(
                barrier, inc=1,
                device_id=(p,), device_id_type=pl.DeviceIdType.MESH,
            )

        def w_copies(s):
            dest = (my_pos + s) % N_DEV
            return [
                pltpu.make_async_copy(
                    w_hbm.at[pl.ds(c * rows, rows), pl.ds(dest * n_loc, n_loc)],
                    w_vmem.at[s % N_WSLOT, pl.ds(c * rows, rows)],
                    w_sems.at[s % N_WSLOT, c],
                )
                for c in range(N_WCHUNK)
            ]

        for s in range(N_WSLOT):
            for cp in w_copies(s):
                cp.start()
        amax_parts = []
        f32_rdmas = []
        for s in range(N_DEV):
            for cp in w_copies(s):
                cp.wait()
            if s + N_WSLOT < N_DEV:
                for cp in w_copies(s + N_WSLOT):
                    cp.start()
            y = jnp.dot(x_ref[:, :], w_vmem[s % N_WSLOT],
                        preferred_element_type=jnp.float32)
            amax_parts.append(jnp.max(jnp.abs(y)))
            blocks[s] = y
            if s == 1:
                pl.semaphore_wait(barrier, N_DEV)
            if 1 <= s < SPLIT:
                dest = (my_pos + s) % N_DEV
                r = pltpu.make_async_remote_copy(
                    src_ref=blocks.at[s],
                    dst_ref=recv_f32.at[s],
                    send_sem=send_sems.at[s],
                    recv_sem=recv_sems.at[s],
                    device_id=(dest,),
                    device_id_type=pl.DeviceIdType.MESH,
                )
                r.start()
                f32_rdmas.append(r)

        my_amax = functools.reduce(jnp.maximum, amax_parts)
        amax_buf[0] = jnp.full((8, 128), my_amax, jnp.float32)
        amax_rdmas = []
        for s in range(1, N_DEV):
            dest = (my_pos + s) % N_DEV
            r = pltpu.make_async_remote_copy(
                src_ref=amax_buf.at[0],
                dst_ref=amax_buf.at[s],
                send_sem=amax_send_sems.at[s],
                recv_sem=amax_recv_sems.at[s],
                device_id=(dest,),
                device_id_type=pl.DeviceIdType.MESH,
            )
            r.start()
            amax_rdmas.append(r)
        for r in amax_rdmas:
            r.wait_recv()
        global_amax = jnp.max(amax_buf[:, :, :])
        scale = global_amax / 127.0

        def quant(y):
            return jnp.clip(jnp.round(y / scale), -127.0, 127.0).astype(jnp.int8)

        i8_rdmas = []
        for s in range(SPLIT, N_DEV):
            send_i8[s] = quant(blocks[s])
            dest = (my_pos + s) % N_DEV
            rdma = pltpu.make_async_remote_copy(
                src_ref=send_i8.at[s],
                dst_ref=recv_i8.at[s],
                send_sem=send_sems.at[s],
                recv_sem=recv_sems.at[s],
                device_id=(dest,),
                device_id_type=pl.DeviceIdType.MESH,
            )
            rdma.start()
            i8_rdmas.append(rdma)

        out_ref[pl.ds(my_pos * m_loc, m_loc), :] = (
            quant(blocks[0]).astype(jnp.float32) * scale)
        for s in range(1, SPLIT):
            f32_rdmas[s - 1].wait_recv()
            src = (my_pos - s) % N_DEV
            out_ref[pl.ds(src * m_loc, m_loc), :] = (
                quant(recv_f32[s]).astype(jnp.float32) * scale)
        for s in range(SPLIT, N_DEV):
            i8_rdmas[s - SPLIT].wait_recv()
            src = (my_pos - s) % N_DEV
            out_ref[pl.ds(src * m_loc, m_loc), :] = (
                recv_i8[s].astype(jnp.float32) * scale)

        for r in f32_rdmas:
            r.wait_send()
        for r in i8_rdmas:
            r.wait_send()
        for r in amax_rdmas:
            r.wait_send()

    return pl.pallas_call(
        body,
        out_shape=jax.ShapeDtypeStruct((m, n_loc), jnp.float32),
        in_specs=[
            pl.BlockSpec(memory_space=pltpu.VMEM),
            pl.BlockSpec(memory_space=pl.ANY),
        ],
        out_specs=pl.BlockSpec(memory_space=pltpu.VMEM),
        scratch_shapes=[
            pltpu.VMEM((N_WSLOT, k, n_loc), jnp.float32),
            pltpu.VMEM((N_DEV, m_loc, n_loc), jnp.float32),
            pltpu.VMEM((SPLIT, m_loc, n_loc), jnp.float32),
            pltpu.VMEM((N_DEV, m_loc, n_loc), jnp.int8),
            pltpu.VMEM((N_DEV, m_loc, n_loc), jnp.int8),
            pltpu.VMEM((N_DEV, 8, 128), jnp.float32),
            pltpu.SemaphoreType.DMA((N_WSLOT, N_WCHUNK)),
            pltpu.SemaphoreType.DMA((N_DEV,)),
            pltpu.SemaphoreType.DMA((N_DEV,)),
            pltpu.SemaphoreType.DMA((N_DEV,)),
            pltpu.SemaphoreType.DMA((N_DEV,)),
        ],
        compiler_params=pltpu.CompilerParams(
            collective_id=0,
            vmem_limit_bytes=100 * 1024 * 1024,
        ),
    )(x, w_mat)
